# baseline (speedup 1.0000x reference)
import functools

import jax
import jax.numpy as jnp
from jax.experimental import pallas as pl
from jax.experimental.pallas import tpu as pltpu


def _gap4d_kernel(x_ref, o_ref, *, inv_hw):
    x = x_ref[...]
    s = jnp.sum(x, axis=2, keepdims=True)
    s = jnp.sum(s, axis=3, keepdims=True, dtype=jnp.float32)
    o_ref[...] = (s * inv_hw).astype(o_ref.dtype)


def kernel(x):
    N, C, H, W = x.shape
    nb = 2
    while N % nb != 0:
        nb //= 2
    grid = N // nb

    return pl.pallas_call(
        functools.partial(_gap4d_kernel, inv_hw=1.0 / (H * W)),
        out_shape=jax.ShapeDtypeStruct((N, C, 1, 1), x.dtype),
        grid=(grid,),
        in_specs=[pl.BlockSpec((nb, C, H, W), lambda i: (i, 0, 0, 0))],
        out_specs=pl.BlockSpec((nb, C, 1, 1), lambda i: (i, 0, 0, 0)),
        compiler_params=pltpu.CompilerParams(
            dimension_semantics=("parallel",),
            vmem_limit_bytes=64 * 1024 * 1024,
        ),
        cost_estimate=pl.CostEstimate(
            flops=N * C * H * W,
            transcendentals=0,
            bytes_accessed=N * C * H * W * x.dtype.itemsize + N * C * x.dtype.itemsize,
        ),
    )(x)


# speedup vs baseline: 19.7595x; 19.7595x over previous
import functools

import jax
import jax.numpy as jnp
from jax.experimental import pallas as pl
from jax.experimental.pallas import tpu as pltpu


def _gap_nhwc_kernel(x_ref, o_ref, *, inv_hw):
    x = x_ref[...]
    s = jnp.sum(x, axis=1)
    o_ref[...] = (s * inv_hw).astype(o_ref.dtype)


def kernel(x):
    N, C, H, W = x.shape
    hw = H * W

    xt = jnp.transpose(x, (0, 2, 3, 1)).reshape(N, hw, C)

    bn = 8
    while N % bn != 0:
        bn //= 2
    grid = N // bn

    out = pl.pallas_call(
        functools.partial(_gap_nhwc_kernel, inv_hw=1.0 / hw),
        out_shape=jax.ShapeDtypeStruct((N, C), x.dtype),
        grid=(grid,),
        in_specs=[pl.BlockSpec((bn, hw, C), lambda i: (i, 0, 0))],
        out_specs=pl.BlockSpec((bn, C), lambda i: (i, 0)),
        compiler_params=pltpu.CompilerParams(
            dimension_semantics=("parallel",),
            vmem_limit_bytes=64 * 1024 * 1024,
        ),
        cost_estimate=pl.CostEstimate(
            flops=N * C * hw,
            transcendentals=0,
            bytes_accessed=N * C * hw * x.dtype.itemsize + N * C * x.dtype.itemsize,
        ),
    )(xt)

    return out.reshape(N, C, 1, 1)


# speedup vs baseline: 21.6649x; 1.0964x over previous
import functools

import jax
import jax.numpy as jnp
from jax.experimental import pallas as pl
from jax.experimental.pallas import tpu as pltpu


def _gap_nhwc_kernel(x_ref, o_ref, *, inv_hw):
    x = x_ref[...]
    s = jnp.sum(x, axis=1)
    o_ref[...] = (s * inv_hw).astype(o_ref.dtype)


def kernel(x):
    N, C, H, W = x.shape
    hw = H * W

    xt = jnp.transpose(x, (0, 2, 3, 1)).reshape(N, hw, C)

    bn = 16
    while N % bn != 0:
        bn //= 2
    grid = N // bn

    out = pl.pallas_call(
        functools.partial(_gap_nhwc_kernel, inv_hw=1.0 / hw),
        out_shape=jax.ShapeDtypeStruct((N, C), x.dtype),
        grid=(grid,),
        in_specs=[pl.BlockSpec((bn, hw, C), lambda i: (i, 0, 0))],
        out_specs=pl.BlockSpec((bn, C), lambda i: (i, 0)),
        compiler_params=pltpu.CompilerParams(
            dimension_semantics=("parallel",),
            vmem_limit_bytes=64 * 1024 * 1024,
        ),
        cost_estimate=pl.CostEstimate(
            flops=N * C * hw,
            transcendentals=0,
            bytes_accessed=N * C * hw * x.dtype.itemsize + N * C * x.dtype.itemsize,
        ),
    )(xt)

    return out.reshape(N, C, 1, 1)
